# baseline (speedup 1.0000x reference)
import functools

import jax
import jax.numpy as jnp
import numpy as np
from jax import lax
from jax.experimental import pallas as pl
from jax.experimental.pallas import tpu as pltpu
from jax.experimental.pallas import tpu_sc as plsc

BEV_X_RANGE = (-40.0, 40.0)
BEV_Y_RANGE = (0.0, 80.0)
BETA = 1.0
NPX = 10

NC, NS, L = 2, 16, 16
NW = NC * NS


def _make_kernel(B, D, H, W):
    ndet = B * D
    dets_per_w = ndet // NW
    pts_per_det = NPX * NPX
    pts_per_w = dets_per_w * pts_per_det
    nvreg = pts_per_w // L
    nrows = dets_per_w * NPX
    npts = ndet * pts_per_det
    half_rows = nrows // 2
    half_vreg = nvreg // 2

    mesh = plsc.VectorSubcoreMesh(core_axis_name="c", subcore_axis_name="s")

    PIDX = 0
    XL = PIDX + 64
    RK4 = XL + 16
    RT = RK4 + nrows
    KK = RT + nrows
    ROW = KK + pts_per_w
    TJ = ROW + pts_per_w
    LUT_LEN = TJ + pts_per_w

    XA = nrows

    @functools.partial(
        pl.kernel,
        out_type=(
            jax.ShapeDtypeStruct((5 * npts,), jnp.float32),
            jax.ShapeDtypeStruct((npts,), jnp.float32),
        ),
        mesh=mesh,
        compiler_params=pltpu.CompilerParams(needs_layout_passes=False),
        scratch_types=[
            pltpu.VMEM((dets_per_w * 4 + 16,), jnp.float32),
            pltpu.VMEM((LUT_LEN,), jnp.int32),
            pltpu.VMEM((nrows,), jnp.int32),
            pltpu.VMEM((nrows + 32,), jnp.float32),
            pltpu.VMEM((2 * nrows, W), jnp.float32),
            pltpu.VMEM((6 * pts_per_w,), jnp.float32),
            pltpu.SemaphoreType.DMA,
            pltpu.SemaphoreType.DMA,
        ],
    )
    def k(bbi_hbm, depth_hbm, lv_hbm, lut_hbm,
          pts_hbm, conf_hbm,
          bb_v, lut_v, rowidx_v, fprec_v, drl_v, po_v,
          semA, semB):
        wid = lax.axis_index("s") * NC + lax.axis_index("c")
        det0 = wid * dets_per_w
        b = wid // (NW // B)

        in1 = pltpu.async_copy(bbi_hbm.at[pl.ds(det0 * 4, dets_per_w * 4)],
                               bb_v.at[pl.ds(0, dets_per_w * 4)], semA)
        in2 = pltpu.async_copy(bbi_hbm.at[pl.ds(ndet * 4, 16)],
                               bb_v.at[pl.ds(dets_per_w * 4, 16)], semA)
        in3 = pltpu.async_copy(lut_hbm, lut_v, semA)
        in1.wait()
        in2.wait()
        in3.wait()

        wmax = jnp.float32(W - 1)
        hmax = jnp.float32(H - 1)

        for n in range(nrows // L):
            rk4 = lut_v[pl.ds(RK4 + n * L, L)]
            rt = plsc.bitcast(lut_v[pl.ds(RT + n * L, L)], jnp.float32)
            y1 = plsc.load_gather(bb_v, [rk4 + 1])
            y2 = plsc.load_gather(bb_v, [rk4 + 3])
            y1c = jnp.clip(y1, 0.0, hmax)
            y2c = jnp.clip(y2, 0.0, hmax)
            v = y1c + rt * (y2c - y1c)
            vi = jnp.clip(v.astype(jnp.int32), 0, H - 1)
            fprec_v[pl.ds(n * L, L)] = v
            rowidx_v[pl.ds(n * L, L)] = b * H + vi

        cpA1 = pltpu.async_copy(depth_hbm.at[rowidx_v.at[pl.ds(0, half_rows)]],
                                drl_v.at[pl.ds(0, half_rows)], semA)
        cpA2 = pltpu.async_copy(lv_hbm.at[rowidx_v.at[pl.ds(0, half_rows)]],
                                drl_v.at[pl.ds(nrows, half_rows)], semA)
        cpB1 = pltpu.async_copy(
            depth_hbm.at[rowidx_v.at[pl.ds(half_rows, half_rows)]],
            drl_v.at[pl.ds(half_rows, half_rows)], semB)
        cpB2 = pltpu.async_copy(
            lv_hbm.at[rowidx_v.at[pl.ds(half_rows, half_rows)]],
            drl_v.at[pl.ds(nrows + half_rows, half_rows)], semB)

        xl = lut_v[pl.ds(XL, L)]
        x1 = plsc.load_gather(bb_v, [xl])
        x2 = plsc.load_gather(bb_v, [xl + 2])
        x1c = jnp.clip(x1, 0.0, wmax)
        x2c = jnp.clip(x2, 0.0, wmax)
        fprec_v[pl.ds(XA, L)] = x1c
        fprec_v[pl.ds(XA + L, L)] = x2c - x1c

        fxv = plsc.load_gather(bb_v, [lut_v[pl.ds(PIDX + 0 * L, L)]])
        fyv = plsc.load_gather(bb_v, [lut_v[pl.ds(PIDX + 1 * L, L)]])
        cxv = plsc.load_gather(bb_v, [lut_v[pl.ds(PIDX + 2 * L, L)]])
        cyv = plsc.load_gather(bb_v, [lut_v[pl.ds(PIDX + 3 * L, L)]])
        rfxv = jnp.float32(1.0) / fxv
        rfyv = jnp.float32(1.0) / fyv

        def point_vreg(n):
            base = n * L
            kk = lut_v[pl.ds(KK + base, L)]
            row = lut_v[pl.ds(ROW + base, L)]
            tj = plsc.bitcast(lut_v[pl.ds(TJ + base, L)], jnp.float32)
            xa = plsc.load_gather(fprec_v, [kk])
            xd = plsc.load_gather(fprec_v, [kk + L])
            u = xa + tj * xd
            v = plsc.load_gather(fprec_v, [row])
            ui = jnp.clip(u.astype(jnp.int32), 0, W - 1)
            dep = plsc.load_gather(drl_v, [row, ui])
            lv = plsc.load_gather(drl_v, [row + nrows, ui])
            conf = jnp.clip(jnp.exp(-BETA * lv), 0.0, 1.0)
            x_cam = (u - cxv) * dep * rfxv
            y_cam = (v - cyv) * dep * rfyv
            x_r = dep
            y_r = -x_cam
            z_r = -y_cam
            mask = ((dep > 0.5)
                    & (x_r > BEV_Y_RANGE[0]) & (x_r < BEV_Y_RANGE[1])
                    & (y_r > BEV_X_RANGE[0]) & (y_r < BEV_X_RANGE[1]))
            mf = jnp.where(mask, jnp.float32(1.0), jnp.float32(0.0))
            po_v[pl.ds(base, L)] = x_r * mf
            po_v[pl.ds(pts_per_w + base, L)] = y_r * mf
            po_v[pl.ds(2 * pts_per_w + base, L)] = z_r * mf
            po_v[pl.ds(3 * pts_per_w + base, L)] = jnp.zeros((L,), jnp.float32)
            po_v[pl.ds(4 * pts_per_w + base, L)] = jnp.float32(10.0) * mf
            po_v[pl.ds(5 * pts_per_w + base, L)] = conf * mf

        def body(n, carry):
            point_vreg(n)
            return carry

        cpA1.wait()
        cpA2.wait()
        lax.fori_loop(0, half_vreg, body, 0, unroll=2)
        cpB1.wait()
        cpB2.wait()
        lax.fori_loop(half_vreg, nvreg, body, 0, unroll=2)

        base_out = wid * pts_per_w
        outs = []
        for c in range(5):
            outs.append(pltpu.async_copy(
                po_v.at[pl.ds(c * pts_per_w, pts_per_w)],
                pts_hbm.at[pl.ds(c * npts + base_out, pts_per_w)], semB))
        outs.append(pltpu.async_copy(
            po_v.at[pl.ds(5 * pts_per_w, pts_per_w)],
            conf_hbm.at[pl.ds(base_out, pts_per_w)], semB))
        for cp in outs:
            cp.wait()

    return k


def kernel(images, depth_map, log_var_map, bboxes, intrinsic):
    del images
    B, _, H, W = depth_map.shape
    D = bboxes.shape[1]
    ndet = B * D
    pts_per_w = (ndet // NW) * NPX * NPX
    nrows = (ndet // NW) * NPX
    ndets_w = nrows // NPX

    depth_rows = depth_map.reshape(B * H, W)
    lv_rows = log_var_map.reshape(B * H, W)
    bbi = jnp.concatenate([
        bboxes.reshape(ndet * 4),
        intrinsic.reshape(9),
        jnp.zeros((7,), jnp.float32),
    ])

    t = np.linspace(0.0, 1.0, NPX).astype(np.float32)
    lr = np.arange(nrows)
    lp = np.arange(pts_per_w)
    xlane = np.minimum(np.arange(16), ndets_w - 1) * 4
    ioff = ndet // NW * 4
    lut = np.concatenate([
        (np.repeat(np.array([0, 4, 2, 5]), 16) + ioff).astype(np.int32),
        xlane.astype(np.int32),
        ((lr // NPX) * 4).astype(np.int32),
        t[lr % NPX].view(np.int32),
        ((lp // (NPX * NPX)) + nrows).astype(np.int32),
        ((lp // (NPX * NPX)) * NPX + (lp // NPX) % NPX).astype(np.int32),
        t[lp % NPX].view(np.int32),
    ])
    lut = jnp.asarray(lut)

    k = _make_kernel(B, D, H, W)
    pts5, conf = k(bbi, depth_rows, lv_rows, lut)
    return pts5.reshape(5, ndet * NPX * NPX).T, conf


# speedup vs baseline: 1.0087x; 1.0087x over previous
import functools

import jax
import jax.numpy as jnp
import numpy as np
from jax import lax
from jax.experimental import pallas as pl
from jax.experimental.pallas import tpu as pltpu
from jax.experimental.pallas import tpu_sc as plsc

BEV_X_RANGE = (-40.0, 40.0)
BEV_Y_RANGE = (0.0, 80.0)
BETA = 1.0
NPX = 10

NC, NS, L = 2, 16, 16
NW = NC * NS


def _make_kernel(B, D, H, W):
    ndet = B * D
    dets_per_w = ndet // NW
    pts_per_det = NPX * NPX
    pts_per_w = dets_per_w * pts_per_det
    nvreg = pts_per_w // L
    nrows = dets_per_w * NPX
    npts = ndet * pts_per_det
    half_rows = nrows // 2
    half_vreg = nvreg // 2

    mesh = plsc.VectorSubcoreMesh(core_axis_name="c", subcore_axis_name="s")

    PIDX = 0
    XL = PIDX + 64
    RK4 = XL + 16
    RT = RK4 + nrows
    KK = RT + nrows
    ROW = KK + pts_per_w
    TJ = ROW + pts_per_w
    LUT_LEN = TJ + pts_per_w

    XA = nrows

    @functools.partial(
        pl.kernel,
        out_type=(
            jax.ShapeDtypeStruct((5 * npts,), jnp.float32),
            jax.ShapeDtypeStruct((npts,), jnp.float32),
        ),
        mesh=mesh,
        compiler_params=pltpu.CompilerParams(needs_layout_passes=False),
        scratch_types=[
            pltpu.VMEM((dets_per_w * 4 + 16,), jnp.float32),
            pltpu.VMEM((LUT_LEN,), jnp.int32),
            pltpu.VMEM((nrows,), jnp.int32),
            pltpu.VMEM((nrows + 32,), jnp.float32),
            pltpu.VMEM((2 * nrows, W), jnp.float32),
            pltpu.VMEM((6 * pts_per_w,), jnp.float32),
            pltpu.SemaphoreType.DMA,
            pltpu.SemaphoreType.DMA,
        ],
    )
    def k(bbi_hbm, depth_hbm, lv_hbm, lut_hbm,
          pts_hbm, conf_hbm,
          bb_v, lut_v, rowidx_v, fprec_v, drl_v, po_v,
          semA, semB):
        wid = lax.axis_index("s") * NC + lax.axis_index("c")
        det0 = wid * dets_per_w
        b = wid // (NW // B)

        in1 = pltpu.async_copy(bbi_hbm.at[pl.ds(det0 * 4, dets_per_w * 4)],
                               bb_v.at[pl.ds(0, dets_per_w * 4)], semA)
        in2 = pltpu.async_copy(bbi_hbm.at[pl.ds(ndet * 4, 16)],
                               bb_v.at[pl.ds(dets_per_w * 4, 16)], semA)
        in3 = pltpu.async_copy(lut_hbm, lut_v, semA)
        in1.wait()
        in2.wait()
        in3.wait()

        wmax = jnp.float32(W - 1)
        hmax = jnp.float32(H - 1)

        for n in range(nrows // L):
            rk4 = lut_v[pl.ds(RK4 + n * L, L)]
            rt = plsc.bitcast(lut_v[pl.ds(RT + n * L, L)], jnp.float32)
            y1 = plsc.load_gather(bb_v, [rk4 + 1])
            y2 = plsc.load_gather(bb_v, [rk4 + 3])
            y1c = jnp.clip(y1, 0.0, hmax)
            y2c = jnp.clip(y2, 0.0, hmax)
            v = y1c + rt * (y2c - y1c)
            vi = jnp.clip(v.astype(jnp.int32), 0, H - 1)
            fprec_v[pl.ds(n * L, L)] = v
            rowidx_v[pl.ds(n * L, L)] = b * H + vi

        cpA1 = pltpu.async_copy(depth_hbm.at[rowidx_v.at[pl.ds(0, half_rows)]],
                                drl_v.at[pl.ds(0, half_rows)], semA)
        cpA2 = pltpu.async_copy(lv_hbm.at[rowidx_v.at[pl.ds(0, half_rows)]],
                                drl_v.at[pl.ds(nrows, half_rows)], semA)
        cpB1 = pltpu.async_copy(
            depth_hbm.at[rowidx_v.at[pl.ds(half_rows, half_rows)]],
            drl_v.at[pl.ds(half_rows, half_rows)], semB)
        cpB2 = pltpu.async_copy(
            lv_hbm.at[rowidx_v.at[pl.ds(half_rows, half_rows)]],
            drl_v.at[pl.ds(nrows + half_rows, half_rows)], semB)

        xl = lut_v[pl.ds(XL, L)]
        x1 = plsc.load_gather(bb_v, [xl])
        x2 = plsc.load_gather(bb_v, [xl + 2])
        x1c = jnp.clip(x1, 0.0, wmax)
        x2c = jnp.clip(x2, 0.0, wmax)
        fprec_v[pl.ds(XA, L)] = x1c
        fprec_v[pl.ds(XA + L, L)] = x2c - x1c

        fxv = plsc.load_gather(bb_v, [lut_v[pl.ds(PIDX + 0 * L, L)]])
        fyv = plsc.load_gather(bb_v, [lut_v[pl.ds(PIDX + 1 * L, L)]])
        cxv = plsc.load_gather(bb_v, [lut_v[pl.ds(PIDX + 2 * L, L)]])
        cyv = plsc.load_gather(bb_v, [lut_v[pl.ds(PIDX + 3 * L, L)]])
        rfxv = jnp.float32(1.0) / fxv
        rfyv = jnp.float32(1.0) / fyv

        def point_vreg(n):
            base = n * L
            kk = lut_v[pl.ds(KK + base, L)]
            row = lut_v[pl.ds(ROW + base, L)]
            tj = plsc.bitcast(lut_v[pl.ds(TJ + base, L)], jnp.float32)
            xa = plsc.load_gather(fprec_v, [kk])
            xd = plsc.load_gather(fprec_v, [kk + L])
            u = xa + tj * xd
            v = plsc.load_gather(fprec_v, [row])
            ui = jnp.clip(u.astype(jnp.int32), 0, W - 1)
            dep = plsc.load_gather(drl_v, [row, ui])
            lv = plsc.load_gather(drl_v, [row + nrows, ui])
            conf = jnp.clip(jnp.exp(-BETA * lv), 0.0, 1.0)
            x_cam = (u - cxv) * dep * rfxv
            y_cam = (v - cyv) * dep * rfyv
            x_r = dep
            y_r = -x_cam
            z_r = -y_cam
            mask = ((dep > 0.5)
                    & (x_r > BEV_Y_RANGE[0]) & (x_r < BEV_Y_RANGE[1])
                    & (y_r > BEV_X_RANGE[0]) & (y_r < BEV_X_RANGE[1]))
            mf = jnp.where(mask, jnp.float32(1.0), jnp.float32(0.0))
            po_v[pl.ds(base, L)] = x_r * mf
            po_v[pl.ds(pts_per_w + base, L)] = y_r * mf
            po_v[pl.ds(2 * pts_per_w + base, L)] = z_r * mf
            po_v[pl.ds(3 * pts_per_w + base, L)] = jnp.zeros((L,), jnp.float32)
            po_v[pl.ds(4 * pts_per_w + base, L)] = jnp.float32(10.0) * mf
            po_v[pl.ds(5 * pts_per_w + base, L)] = conf * mf

        def body(n, carry):
            point_vreg(n)
            return carry

        cpA1.wait()
        cpA2.wait()
        lax.fori_loop(0, half_vreg, body, 0)
        cpB1.wait()
        cpB2.wait()
        lax.fori_loop(half_vreg, nvreg, body, 0)

        base_out = wid * pts_per_w
        outs = []
        for c in range(5):
            outs.append(pltpu.async_copy(
                po_v.at[pl.ds(c * pts_per_w, pts_per_w)],
                pts_hbm.at[pl.ds(c * npts + base_out, pts_per_w)], semB))
        outs.append(pltpu.async_copy(
            po_v.at[pl.ds(5 * pts_per_w, pts_per_w)],
            conf_hbm.at[pl.ds(base_out, pts_per_w)], semB))
        for cp in outs:
            cp.wait()

    return k


def kernel(images, depth_map, log_var_map, bboxes, intrinsic):
    del images
    B, _, H, W = depth_map.shape
    D = bboxes.shape[1]
    ndet = B * D
    pts_per_w = (ndet // NW) * NPX * NPX
    nrows = (ndet // NW) * NPX
    ndets_w = nrows // NPX

    depth_rows = depth_map.reshape(B * H, W)
    lv_rows = log_var_map.reshape(B * H, W)
    bbi = jnp.concatenate([
        bboxes.reshape(ndet * 4),
        intrinsic.reshape(9),
        jnp.zeros((7,), jnp.float32),
    ])

    t = np.linspace(0.0, 1.0, NPX).astype(np.float32)
    lr = np.arange(nrows)
    lp = np.arange(pts_per_w)
    xlane = np.minimum(np.arange(16), ndets_w - 1) * 4
    ioff = ndet // NW * 4
    lut = np.concatenate([
        (np.repeat(np.array([0, 4, 2, 5]), 16) + ioff).astype(np.int32),
        xlane.astype(np.int32),
        ((lr // NPX) * 4).astype(np.int32),
        t[lr % NPX].view(np.int32),
        ((lp // (NPX * NPX)) + nrows).astype(np.int32),
        ((lp // (NPX * NPX)) * NPX + (lp // NPX) % NPX).astype(np.int32),
        t[lp % NPX].view(np.int32),
    ])
    lut = jnp.asarray(lut)

    k = _make_kernel(B, D, H, W)
    pts5, conf = k(bbi, depth_rows, lv_rows, lut)
    return pts5.reshape(5, ndet * NPX * NPX).T, conf
